# baseline (speedup 1.0000x reference)
import functools

import jax
import jax.numpy as jnp
from jax import lax
from jax.experimental import pallas as pl
from jax.experimental.pallas import tpu as pltpu
from jax.experimental.pallas import tpu_sc as plsc

_S = 30.0
_M = 0.5
_STD = 0.0125
_SHIFT = 30.0
_LOG2E = 1.4426950408889634
_A = _S * _LOG2E
_BB = _SHIFT * _LOG2E

_K_SC = 61440
_NC, _NS = 2, 16
_NW = _NC * _NS
_CHUNK = 4096
_VEC = 16
_CB = 4096
_TC_R = 256


def _tc_kernel(x_ref, out_ref, sum_ref, *, n_cols, blk_k, cb_off):
    cb = pl.program_id(1)
    ncb = pl.num_programs(1)

    @pl.when(cb == 0)
    def _init():
        sum_ref[...] = jnp.zeros_like(sum_ref)

    x = x_ref[...]
    r, k = x.shape
    e = jnp.exp2(x * _A - _BB)

    @pl.when(cb != ncb - 1)
    def _body():
        sum_ref[...] += jnp.sum(e, axis=1, keepdims=True)

    @pl.when(cb == ncb - 1)
    def _last():
        col = jax.lax.broadcasted_iota(jnp.int32, (r, k), 1) \
            + (cb + cb_off) * blk_k
        sum_ref[...] += jnp.sum(jnp.where(col < n_cols, e, 0.0),
                                axis=1, keepdims=True)
        out_ref[...] = sum_ref[...]


def _epilogue_kernel(tc_ref, sc_ref, xlab_ref, cosm_ref, sinm_ref, out_ref):
    xl = xlab_ref[...]
    e_lab = jnp.exp2(xl * _A - _BB)
    xlc = jnp.clip(xl, -1.0 + 1e-7, 1.0 - 1e-7)
    sin_theta = jnp.sqrt(jnp.maximum(1.0 - xlc * xlc, 0.0))
    mprime = (xlc * cosm_ref[...] - sin_theta * sinm_ref[...]) * _S
    total = tc_ref[...] + sc_ref[...] - e_lab \
        + jnp.exp2(mprime * _LOG2E - _BB)
    out_ref[...] = jnp.log(total) + _SHIFT - mprime


def _make_sc_part(n_rows):
    rpw = n_rows // _NW
    n_groups = rpw // 8
    chunks_per_group = _K_SC // _CHUNK
    n_steps = n_groups * chunks_per_group
    vecs = _CHUNK // _VEC
    mesh = plsc.VectorSubcoreMesh(core_axis_name="c", subcore_axis_name="s")

    @functools.partial(
        pl.kernel,
        mesh=mesh,
        out_type=(
            jax.ShapeDtypeStruct((n_rows,), jnp.float32),
            jax.ShapeDtypeStruct((n_rows,), jnp.float32),
        ),
        scratch_types=[
            pltpu.VMEM((2, 8, _CHUNK), jnp.float32),
            pltpu.VMEM((rpw,), jnp.float32),
            pltpu.VMEM((rpw,), jnp.float32),
            pltpu.VMEM((rpw,), jnp.float32),
            pltpu.VMEM((rpw, 8, 128), jnp.float32),
            pltpu.VMEM((rpw,), jnp.float32),
            pltpu.VMEM((rpw,), jnp.float32),
            pltpu.SemaphoreType.DMA((2,)),
            pltpu.SemaphoreType.DMA,
        ],
        compiler_params=pltpu.CompilerParams(needs_layout_passes=False),
    )
    def _sc(x_hbm, c0_hbm, v0_hbm, lane_hbm, sums_hbm, xlab_hbm, buf,
            c0v, v0v, lanev, ltile, sstage, xstage, sems, lsem):
        wid = lax.axis_index("s") * _NC + lax.axis_index("c")
        row0 = wid * rpw
        iota = lax.iota(jnp.int32, _VEC)

        pltpu.sync_copy(c0_hbm.at[pl.ds(row0, rpw)], c0v)
        pltpu.sync_copy(v0_hbm.at[pl.ds(row0, rpw)], v0v)
        pltpu.sync_copy(lane_hbm.at[pl.ds(row0, rpw)], lanev)

        def _scalar_at(ref, r):
            half = ref[pl.ds((r // _VEC) * _VEC, _VEC)]
            return jnp.sum(jnp.where(iota == lax.rem(r, _VEC), half, 0.0))

        @pl.loop(0, rpw)
        def _fire(r):
            c0 = pl.multiple_of(_scalar_at(c0v, r).astype(jnp.int32), 128)
            g8 = row0 + (r // 8) * 8
            pltpu.async_copy(x_hbm.at[pl.ds(g8, 8), pl.ds(c0, 128)],
                             ltile.at[r], lsem)

        pltpu.async_copy(x_hbm.at[pl.ds(row0, 8), pl.ds(0, _CHUNK)],
                         buf.at[0], sems.at[0])

        init = (jnp.zeros((_VEC,), jnp.float32),
                jnp.zeros((_VEC,), jnp.float32))

        @pl.loop(0, n_steps, init_carry=init)
        def svecs(t, sv):
            s0, s1 = sv
            slot = lax.rem(t, 2)
            g = t // chunks_per_group
            k = lax.rem(t, chunks_per_group)

            @pl.when(t + 1 < n_steps)
            def _prefetch():
                t2 = t + 1
                nslot = lax.rem(t2, 2)
                g2 = t2 // chunks_per_group
                k2 = lax.rem(t2, chunks_per_group)
                pltpu.async_copy(
                    x_hbm.at[pl.ds(row0 + g2 * 8, 8),
                             pl.ds(k2 * _CHUNK, _CHUNK)],
                    buf.at[nslot], sems.at[nslot])

            pltpu.make_async_copy(
                x_hbm.at[pl.ds(row0, 8), pl.ds(0, _CHUNK)],
                buf.at[slot], sems.at[slot]).wait()

            for r in range(8):
                zero2 = (jnp.zeros((_VEC,), jnp.float32),
                         jnp.zeros((_VEC,), jnp.float32))

                def _acc_body(i, c, _slot=slot, _r=r):
                    a0, a1 = c
                    v0 = buf[_slot, _r, pl.ds(i * _VEC, _VEC)]
                    v1 = buf[_slot, _r, pl.ds((i + 1) * _VEC, _VEC)]
                    return (a0 + jnp.exp(v0 * _S - _SHIFT),
                            a1 + jnp.exp(v1 * _S - _SHIFT))

                a0, a1 = plsc.parallel_loop(
                    0, vecs, 2, unroll=4, carry=zero2)(_acc_body)
                row_sum = jnp.sum(a0 + a1)
                ridx = g * 8 + r
                lane_hit = iota == lax.rem(ridx, _VEC)
                in0 = ridx // _VEC == 0
                s0 = jnp.where(in0 & lane_hit, s0 + row_sum, s0)
                s1 = jnp.where(jnp.logical_not(in0) & lane_hit,
                               s1 + row_sum, s1)
            return (s0, s1)

        sstage[pl.ds(0, _VEC)] = svecs[0]
        sstage[pl.ds(_VEC, _VEC)] = svecs[1]
        pltpu.sync_copy(sstage, sums_hbm.at[pl.ds(row0, rpw)])

        @pl.loop(0, rpw)
        def _drain(r):
            pltpu.make_async_copy(x_hbm.at[pl.ds(0, 8), pl.ds(0, 128)],
                                  ltile.at[r], lsem).wait()

        zerox = (jnp.zeros((_VEC,), jnp.float32),
                 jnp.zeros((_VEC,), jnp.float32))

        @pl.loop(0, rpw, init_carry=zerox)
        def xvecs(r, xv):
            x0, x1 = xv
            v0 = _scalar_at(v0v, r).astype(jnp.int32)
            lane = _scalar_at(lanev, r).astype(jnp.int32)
            v = ltile[r, lax.rem(r, 8), pl.ds(v0, _VEC)]
            xl_r = jnp.sum(jnp.where(iota == lane, v, 0.0))
            lane_hit = iota == lax.rem(r, _VEC)
            in0 = r // _VEC == 0
            x0 = jnp.where(in0 & lane_hit, xl_r, x0)
            x1 = jnp.where(jnp.logical_not(in0) & lane_hit, xl_r, x1)
            return (x0, x1)

        xstage[pl.ds(0, _VEC)] = xvecs[0]
        xstage[pl.ds(_VEC, _VEC)] = xvecs[1]
        pltpu.sync_copy(xstage, xlab_hbm.at[pl.ds(row0, rpw)])

    return _sc


@jax.jit
def kernel(input, label):
    b, c = input.shape
    cb_off = _K_SC // _CB
    n_cb = pl.cdiv(c - _K_SC, _CB)
    n_rb = b // _TC_R

    margin = _M + _STD * jax.random.normal(jax.random.key(42), (b,),
                                           dtype=jnp.float32)
    valid = label != -1
    margin = jnp.where(valid, margin, 0.0)
    safe_label = jnp.where(valid, label, 0).astype(jnp.int32)
    cos_m = jnp.cos(margin)[:, None]
    sin_m = jnp.sin(margin)[:, None]

    c0_f = ((safe_label // 128) * 128).astype(jnp.float32)
    v0_f = (((safe_label % 128) // _VEC) * _VEC).astype(jnp.float32)
    lane_f = (safe_label % _VEC).astype(jnp.float32)
    sc_sums, sc_xlab = _make_sc_part(b)(input, c0_f, v0_f, lane_f)

    tc_sums = pl.pallas_call(
        functools.partial(_tc_kernel, n_cols=c, blk_k=_CB, cb_off=cb_off),
        grid=(n_rb, n_cb),
        in_specs=[
            pl.BlockSpec((_TC_R, _CB), lambda rb, cb: (rb, cb + cb_off)),
        ],
        out_specs=pl.BlockSpec((_TC_R, 1), lambda rb, cb: (rb, 0)),
        out_shape=jax.ShapeDtypeStruct((b, 1), jnp.float32),
        scratch_shapes=[pltpu.VMEM((_TC_R, 1), jnp.float32)],
        compiler_params=pltpu.CompilerParams(
            dimension_semantics=("parallel", "arbitrary"),
        ),
    )(input)

    losses = pl.pallas_call(
        _epilogue_kernel,
        in_specs=[pl.BlockSpec((b, 1), lambda: (0, 0))] * 5,
        out_specs=pl.BlockSpec((b, 1), lambda: (0, 0)),
        out_shape=jax.ShapeDtypeStruct((b, 1), jnp.float32),
    )(tc_sums, sc_sums[:, None], sc_xlab[:, None], cos_m, sin_m)

    return jnp.mean(losses)


# speedup vs baseline: 2.7425x; 2.7425x over previous
import functools

import jax
import jax.numpy as jnp
from jax.experimental import pallas as pl
from jax.experimental.pallas import tpu as pltpu

_S = 30.0
_M = 0.5
_STD = 0.0125
_SHIFT = 30.0
_LOG2E = 1.4426950408889634
_A = _S * _LOG2E
_BB = _SHIFT * _LOG2E

_CK = 2048


def _loss_kernel(label_ref, cosm_ref, sinm_ref, x_ref, out_ref, sum_ref,
                 xl_ref, *, n_classes):
    cb = pl.program_id(0)
    ncb = pl.num_programs(0)

    @pl.when(cb == 0)
    def _init():
        sum_ref[...] = jnp.zeros_like(sum_ref)
        xl_ref[...] = jnp.zeros_like(xl_ref)

    x = x_ref[...]
    k, b = x.shape
    cls = jax.lax.broadcasted_iota(jnp.int32, (k, b), 0) + cb * _CK
    lab = label_ref[...]
    hit = cls == lab
    e = jnp.exp2(x * _A - _BB)
    xl_ref[0:1, :] += jnp.sum(jnp.where(hit, x, 0.0), axis=0, keepdims=True)

    @pl.when(cb != ncb - 1)
    def _body():
        sum_ref[0:1, :] += jnp.sum(jnp.where(hit, 0.0, e), axis=0,
                                   keepdims=True)

    @pl.when(cb == ncb - 1)
    def _last():
        dead = hit | (cls >= n_classes)
        sum_ref[0:1, :] += jnp.sum(jnp.where(dead, 0.0, e), axis=0,
                                   keepdims=True)

        xl = jnp.clip(xl_ref[0:1, :], -1.0 + 1e-7, 1.0 - 1e-7)
        sin_theta = jnp.sqrt(jnp.maximum(1.0 - xl * xl, 0.0))
        mprime = (xl * cosm_ref[...] - sin_theta * sinm_ref[...]) * _S
        total = sum_ref[0:1, :] + jnp.exp2(mprime * _LOG2E - _BB)
        out_ref[...] = jnp.log(total) + _SHIFT - mprime


@jax.jit
def kernel(input, label):
    b, c = input.shape
    xt = input.T
    n_cb = pl.cdiv(c, _CK)

    margin = _M + _STD * jax.random.normal(jax.random.key(42), (b,),
                                           dtype=jnp.float32)
    valid = label != -1
    margin = jnp.where(valid, margin, 0.0)
    safe_label = jnp.where(valid, label, 0).astype(jnp.int32)
    cos_m = jnp.cos(margin)[None, :]
    sin_m = jnp.sin(margin)[None, :]

    losses = pl.pallas_call(
        functools.partial(_loss_kernel, n_classes=c),
        grid=(n_cb,),
        in_specs=[
            pl.BlockSpec((1, b), lambda cb: (0, 0)),
            pl.BlockSpec((1, b), lambda cb: (0, 0)),
            pl.BlockSpec((1, b), lambda cb: (0, 0)),
            pl.BlockSpec((_CK, b), lambda cb: (cb, 0)),
        ],
        out_specs=pl.BlockSpec((1, b), lambda cb: (0, 0)),
        out_shape=jax.ShapeDtypeStruct((1, b), jnp.float32),
        scratch_shapes=[
            pltpu.VMEM((8, b), jnp.float32),
            pltpu.VMEM((8, b), jnp.float32),
        ],
        compiler_params=pltpu.CompilerParams(
            dimension_semantics=("arbitrary",),
        ),
    )(safe_label[None, :], cos_m, sin_m, xt)

    return jnp.mean(losses)


# speedup vs baseline: 3.3358x; 1.2163x over previous
import functools

import jax
import jax.numpy as jnp
from jax import lax
from jax.experimental import pallas as pl
from jax.experimental.pallas import tpu as pltpu
from jax.experimental.pallas import tpu_sc as plsc

_S = 30.0
_M = 0.5
_STD = 0.0125
_SHIFT = 30.0
_LOG2E = 1.4426950408889634
_A = _S * _LOG2E
_BB = _SHIFT * _LOG2E

_K_SC = 40960
_NC, _NS = 2, 16
_NW = _NC * _NS
_CR = 40
_VEC = 16
_CK = 2048


def _tc_kernel(x_ref, out_ref, sum_ref, *, n_classes, cb_off):
    cb = pl.program_id(0)
    ncb = pl.num_programs(0)

    @pl.when(cb == 0)
    def _init():
        sum_ref[...] = jnp.zeros_like(sum_ref)

    x = x_ref[...]
    k, b = x.shape
    e = jnp.exp2(x * _A - _BB)

    @pl.when(cb != ncb - 1)
    def _body():
        sum_ref[0:1, :] += jnp.sum(e, axis=0, keepdims=True)

    @pl.when(cb == ncb - 1)
    def _last():
        cls = jax.lax.broadcasted_iota(jnp.int32, (k, b), 0) \
            + (cb + cb_off) * _CK
        sum_ref[0:1, :] += jnp.sum(jnp.where(cls < n_classes, e, 0.0),
                                   axis=0, keepdims=True)
        out_ref[...] = sum_ref[0:1, :]


def _epilogue_kernel(tc_ref, sc_ref, xlab_ref, cosm_ref, sinm_ref, out_ref):
    xl = xlab_ref[...]
    e_lab = jnp.exp2(xl * _A - _BB)
    xlc = jnp.clip(xl, -1.0 + 1e-7, 1.0 - 1e-7)
    sin_theta = jnp.sqrt(jnp.maximum(1.0 - xlc * xlc, 0.0))
    mprime = (xlc * cosm_ref[...] - sin_theta * sinm_ref[...]) * _S
    total = tc_ref[...] + jnp.sum(sc_ref[...], axis=0, keepdims=True) \
        - e_lab + jnp.exp2(mprime * _LOG2E - _BB)
    out_ref[...] = jnp.log(total) + _SHIFT - mprime


def _make_sc_part(n_batch):
    span = _K_SC // _NW
    n_chunks = span // _CR
    bpw = n_batch // _NW
    nbv = n_batch // _VEC
    mesh = plsc.VectorSubcoreMesh(core_axis_name="c", subcore_axis_name="s")

    @functools.partial(
        pl.kernel,
        mesh=mesh,
        out_type=(
            jax.ShapeDtypeStruct((_NW * n_batch,), jnp.float32),
            jax.ShapeDtypeStruct((n_batch,), jnp.float32),
        ),
        scratch_types=[
            pltpu.VMEM((2, _CR, n_batch), jnp.float32),
            pltpu.VMEM((n_batch,), jnp.float32),
            pltpu.VMEM((bpw,), jnp.float32),
            pltpu.VMEM((bpw,), jnp.float32),
            pltpu.VMEM((bpw, 8, 128), jnp.float32),
            pltpu.VMEM((bpw,), jnp.float32),
            pltpu.SemaphoreType.DMA((2,)),
            pltpu.SemaphoreType.DMA,
        ],
        compiler_params=pltpu.CompilerParams(needs_layout_passes=False),
    )
    def _sc(x_hbm, g8_hbm, sub8_hbm, sums_hbm, xlab_hbm, buf, acc, g8v,
            sub8v, ltile, xstage, sems, lsem):
        wid = lax.axis_index("s") * _NC + lax.axis_index("c")
        r0 = wid * span
        b32 = wid * bpw
        b0 = (b32 // 128) * 128
        iota = lax.iota(jnp.int32, _VEC)

        pltpu.sync_copy(g8_hbm.at[pl.ds(b32, bpw)], g8v)
        pltpu.sync_copy(sub8_hbm.at[pl.ds(b32, bpw)], sub8v)

        def _scalar_at(ref, r):
            half = ref[pl.ds((r // _VEC) * _VEC, _VEC)]
            return jnp.sum(jnp.where(iota == lax.rem(r, _VEC), half, 0.0))

        @pl.loop(0, bpw)
        def _fire(r):
            g8 = pl.multiple_of(_scalar_at(g8v, r).astype(jnp.int32), 8)
            pltpu.async_copy(x_hbm.at[pl.ds(g8, 8), pl.ds(b0, 128)],
                             ltile.at[r], lsem)

        @pl.loop(0, nbv)
        def _zero(j):
            acc[pl.ds(j * _VEC, _VEC)] = jnp.zeros((_VEC,), jnp.float32)

        pltpu.async_copy(
            x_hbm.at[pl.ds(r0, _CR), :], buf.at[0], sems.at[0])

        @pl.loop(0, n_chunks)
        def _chunks(t):
            slot = lax.rem(t, 2)

            @pl.when(t + 1 < n_chunks)
            def _prefetch():
                nslot = lax.rem(t + 1, 2)
                row = pl.multiple_of(r0 + (t + 1) * _CR, 8)
                pltpu.async_copy(x_hbm.at[pl.ds(row, _CR), :],
                                 buf.at[nslot], sems.at[nslot])

            pltpu.make_async_copy(
                x_hbm.at[pl.ds(r0, _CR), :],
                buf.at[slot], sems.at[slot]).wait()

            @pl.loop(0, nbv)
            def _bv(j):
                zero4 = (jnp.zeros((_VEC,), jnp.float32),) * 4

                def _acc_body(r, c):
                    a0, a1, a2, a3 = c
                    col = pl.ds(j * _VEC, _VEC)
                    a0 = a0 + jnp.exp(buf[slot, r, col] * _S - _SHIFT)
                    a1 = a1 + jnp.exp(buf[slot, r + 1, col] * _S - _SHIFT)
                    a2 = a2 + jnp.exp(buf[slot, r + 2, col] * _S - _SHIFT)
                    a3 = a3 + jnp.exp(buf[slot, r + 3, col] * _S - _SHIFT)
                    return (a0, a1, a2, a3)

                a0, a1, a2, a3 = plsc.parallel_loop(
                    0, _CR, 4, unroll=2, carry=zero4)(_acc_body)
                col = pl.ds(j * _VEC, _VEC)
                acc[col] = acc[col] + ((a0 + a1) + (a2 + a3))

        pltpu.sync_copy(acc, sums_hbm.at[pl.ds(wid * n_batch, n_batch)])

        @pl.loop(0, bpw)
        def _drain(r):
            pltpu.make_async_copy(x_hbm.at[pl.ds(0, 8), pl.ds(0, 128)],
                                  ltile.at[r], lsem).wait()

        zerox = (jnp.zeros((_VEC,), jnp.float32),
                 jnp.zeros((_VEC,), jnp.float32))

        @pl.loop(0, bpw, init_carry=zerox)
        def xvecs(r, xv):
            x0, x1 = xv
            sub8 = _scalar_at(sub8v, r).astype(jnp.int32)
            colv = ((b32 + r - b0) // _VEC) * _VEC
            v = ltile[r, sub8, pl.ds(colv, _VEC)]
            lane_hit = iota == lax.rem(r, _VEC)
            xl_r = jnp.sum(jnp.where(lane_hit, v, 0.0))
            in0 = r // _VEC == 0
            x0 = jnp.where(in0 & lane_hit, xl_r, x0)
            x1 = jnp.where(jnp.logical_not(in0) & lane_hit, xl_r, x1)
            return (x0, x1)

        xstage[pl.ds(0, _VEC)] = xvecs[0]
        xstage[pl.ds(_VEC, _VEC)] = xvecs[1]
        pltpu.sync_copy(xstage, xlab_hbm.at[pl.ds(b32, bpw)])

    return _sc


@jax.jit
def kernel(input, label):
    b, c = input.shape
    xt = input.T
    cb_off = _K_SC // _CK
    n_cb = pl.cdiv(c - _K_SC, _CK)

    margin = _M + _STD * jax.random.normal(jax.random.key(42), (b,),
                                           dtype=jnp.float32)
    valid = label != -1
    margin = jnp.where(valid, margin, 0.0)
    safe_label = jnp.where(valid, label, 0).astype(jnp.int32)
    cos_m = jnp.cos(margin)[None, :]
    sin_m = jnp.sin(margin)[None, :]

    g8_f = ((safe_label // 8) * 8).astype(jnp.float32)
    sub8_f = (safe_label % 8).astype(jnp.float32)

    sc_parts, sc_xlab = _make_sc_part(b)(xt, g8_f, sub8_f)

    tc_sums = pl.pallas_call(
        functools.partial(_tc_kernel, n_classes=c, cb_off=cb_off),
        grid=(n_cb,),
        in_specs=[pl.BlockSpec((_CK, b), lambda cb: (cb + cb_off, 0))],
        out_specs=pl.BlockSpec((1, b), lambda cb: (0, 0)),
        out_shape=jax.ShapeDtypeStruct((1, b), jnp.float32),
        scratch_shapes=[pltpu.VMEM((8, b), jnp.float32)],
        compiler_params=pltpu.CompilerParams(
            dimension_semantics=("arbitrary",),
        ),
    )(xt)

    losses = pl.pallas_call(
        _epilogue_kernel,
        in_specs=[
            pl.BlockSpec((1, b), lambda: (0, 0)),
            pl.BlockSpec((_NW, b), lambda: (0, 0)),
            pl.BlockSpec((1, b), lambda: (0, 0)),
            pl.BlockSpec((1, b), lambda: (0, 0)),
            pl.BlockSpec((1, b), lambda: (0, 0)),
        ],
        out_specs=pl.BlockSpec((1, b), lambda: (0, 0)),
        out_shape=jax.ShapeDtypeStruct((1, b), jnp.float32),
    )(tc_sums, sc_parts.reshape(_NW, b), sc_xlab[None, :], cos_m, sin_m)

    return jnp.mean(losses)


# speedup vs baseline: 3.4544x; 1.0355x over previous
import functools

import jax
import jax.numpy as jnp
from jax import lax
from jax.experimental import pallas as pl
from jax.experimental.pallas import tpu as pltpu
from jax.experimental.pallas import tpu_sc as plsc

_S = 30.0
_M = 0.5
_STD = 0.0125
_SHIFT = 30.0
_LOG2E = 1.4426950408889634
_A = _S * _LOG2E
_BB = _SHIFT * _LOG2E

_K_SC = 36864
_NC, _NS = 2, 16
_NW = _NC * _NS
_CR = 32
_VEC = 16
_CK = 2048


def _tc_kernel(x_ref, out_ref, sum_ref, *, n_classes, cb_off):
    cb = pl.program_id(0)
    ncb = pl.num_programs(0)

    @pl.when(cb == 0)
    def _init():
        sum_ref[...] = jnp.zeros_like(sum_ref)

    x = x_ref[...]
    k, b = x.shape
    e = jnp.exp2(x * _A - _BB)

    @pl.when(cb != ncb - 1)
    def _body():
        sum_ref[0:1, :] += jnp.sum(e, axis=0, keepdims=True)

    @pl.when(cb == ncb - 1)
    def _last():
        cls = jax.lax.broadcasted_iota(jnp.int32, (k, b), 0) \
            + (cb + cb_off) * _CK
        sum_ref[0:1, :] += jnp.sum(jnp.where(cls < n_classes, e, 0.0),
                                   axis=0, keepdims=True)
        out_ref[...] = sum_ref[0:1, :]


def _epilogue_kernel(tc_ref, sc_ref, xlab_ref, cosm_ref, sinm_ref, out_ref):
    xl = xlab_ref[...]
    e_lab = jnp.exp2(xl * _A - _BB)
    xlc = jnp.clip(xl, -1.0 + 1e-7, 1.0 - 1e-7)
    sin_theta = jnp.sqrt(jnp.maximum(1.0 - xlc * xlc, 0.0))
    mprime = (xlc * cosm_ref[...] - sin_theta * sinm_ref[...]) * _S
    total = tc_ref[...] + jnp.sum(sc_ref[...], axis=0, keepdims=True) \
        - e_lab + jnp.exp2(mprime * _LOG2E - _BB)
    out_ref[...] = jnp.log(total) + _SHIFT - mprime


def _make_sc_part(n_batch):
    span = _K_SC // _NW
    n_chunks = span // _CR
    bpw = n_batch // _NW
    nbv = n_batch // _VEC
    mesh = plsc.VectorSubcoreMesh(core_axis_name="c", subcore_axis_name="s")

    @functools.partial(
        pl.kernel,
        mesh=mesh,
        out_type=(
            jax.ShapeDtypeStruct((_NW * n_batch,), jnp.float32),
            jax.ShapeDtypeStruct((n_batch,), jnp.float32),
        ),
        scratch_types=[
            pltpu.VMEM((2, _CR, n_batch), jnp.float32),
            pltpu.VMEM((n_batch,), jnp.float32),
            pltpu.VMEM((bpw,), jnp.float32),
            pltpu.VMEM((bpw,), jnp.float32),
            pltpu.VMEM((bpw, 8, 128), jnp.float32),
            pltpu.VMEM((bpw,), jnp.float32),
            pltpu.SemaphoreType.DMA((2,)),
            pltpu.SemaphoreType.DMA,
        ],
        compiler_params=pltpu.CompilerParams(needs_layout_passes=False),
    )
    def _sc(x_hbm, g8_hbm, sub8_hbm, sums_hbm, xlab_hbm, buf, acc, g8v,
            sub8v, ltile, xstage, sems, lsem):
        wid = lax.axis_index("s") * _NC + lax.axis_index("c")
        r0 = wid * span
        b32 = wid * bpw
        b0 = (b32 // 128) * 128
        iota = lax.iota(jnp.int32, _VEC)

        pltpu.sync_copy(g8_hbm.at[pl.ds(b32, bpw)], g8v)
        pltpu.sync_copy(sub8_hbm.at[pl.ds(b32, bpw)], sub8v)

        def _scalar_at(ref, r):
            half = ref[pl.ds((r // _VEC) * _VEC, _VEC)]
            return jnp.sum(jnp.where(iota == lax.rem(r, _VEC), half, 0.0))

        @pl.loop(0, bpw)
        def _fire(r):
            g8 = pl.multiple_of(_scalar_at(g8v, r).astype(jnp.int32), 8)
            pltpu.async_copy(x_hbm.at[pl.ds(g8, 8), pl.ds(b0, 128)],
                             ltile.at[r], lsem)

        @pl.loop(0, nbv)
        def _zero(j):
            acc[pl.ds(j * _VEC, _VEC)] = jnp.zeros((_VEC,), jnp.float32)

        pltpu.async_copy(
            x_hbm.at[pl.ds(r0, _CR), :], buf.at[0], sems.at[0])

        @pl.loop(0, n_chunks)
        def _chunks(t):
            slot = lax.rem(t, 2)

            @pl.when(t + 1 < n_chunks)
            def _prefetch():
                nslot = lax.rem(t + 1, 2)
                row = pl.multiple_of(r0 + (t + 1) * _CR, 8)
                pltpu.async_copy(x_hbm.at[pl.ds(row, _CR), :],
                                 buf.at[nslot], sems.at[nslot])

            pltpu.make_async_copy(
                x_hbm.at[pl.ds(r0, _CR), :],
                buf.at[slot], sems.at[slot]).wait()

            @pl.loop(0, nbv)
            def _bv(j):
                zero4 = (jnp.zeros((_VEC,), jnp.float32),) * 4

                def _acc_body(r, c):
                    a0, a1, a2, a3 = c
                    col = pl.ds(j * _VEC, _VEC)
                    a0 = a0 + jnp.exp(buf[slot, r, col] * _S - _SHIFT)
                    a1 = a1 + jnp.exp(buf[slot, r + 1, col] * _S - _SHIFT)
                    a2 = a2 + jnp.exp(buf[slot, r + 2, col] * _S - _SHIFT)
                    a3 = a3 + jnp.exp(buf[slot, r + 3, col] * _S - _SHIFT)
                    return (a0, a1, a2, a3)

                a0, a1, a2, a3 = plsc.parallel_loop(
                    0, _CR, 4, unroll=2, carry=zero4)(_acc_body)
                col = pl.ds(j * _VEC, _VEC)
                acc[col] = acc[col] + ((a0 + a1) + (a2 + a3))

        pltpu.sync_copy(acc, sums_hbm.at[pl.ds(wid * n_batch, n_batch)])

        @pl.loop(0, bpw)
        def _drain(r):
            pltpu.make_async_copy(x_hbm.at[pl.ds(0, 8), pl.ds(0, 128)],
                                  ltile.at[r], lsem).wait()

        zerox = (jnp.zeros((_VEC,), jnp.float32),
                 jnp.zeros((_VEC,), jnp.float32))

        @pl.loop(0, bpw, init_carry=zerox)
        def xvecs(r, xv):
            x0, x1 = xv
            sub8 = _scalar_at(sub8v, r).astype(jnp.int32)
            colv = ((b32 + r - b0) // _VEC) * _VEC
            v = ltile[r, sub8, pl.ds(colv, _VEC)]
            lane_hit = iota == lax.rem(r, _VEC)
            xl_r = jnp.sum(jnp.where(lane_hit, v, 0.0))
            in0 = r // _VEC == 0
            x0 = jnp.where(in0 & lane_hit, xl_r, x0)
            x1 = jnp.where(jnp.logical_not(in0) & lane_hit, xl_r, x1)
            return (x0, x1)

        xstage[pl.ds(0, _VEC)] = xvecs[0]
        xstage[pl.ds(_VEC, _VEC)] = xvecs[1]
        pltpu.sync_copy(xstage, xlab_hbm.at[pl.ds(b32, bpw)])

    return _sc


@jax.jit
def kernel(input, label):
    b, c = input.shape
    xt = input.T
    cb_off = _K_SC // _CK
    n_cb = pl.cdiv(c - _K_SC, _CK)

    margin = _M + _STD * jax.random.normal(jax.random.key(42), (b,),
                                           dtype=jnp.float32)
    valid = label != -1
    margin = jnp.where(valid, margin, 0.0)
    safe_label = jnp.where(valid, label, 0).astype(jnp.int32)
    cos_m = jnp.cos(margin)[None, :]
    sin_m = jnp.sin(margin)[None, :]

    g8_f = ((safe_label // 8) * 8).astype(jnp.float32)
    sub8_f = (safe_label % 8).astype(jnp.float32)

    sc_parts, sc_xlab = _make_sc_part(b)(xt, g8_f, sub8_f)

    tc_sums = pl.pallas_call(
        functools.partial(_tc_kernel, n_classes=c, cb_off=cb_off),
        grid=(n_cb,),
        in_specs=[pl.BlockSpec((_CK, b), lambda cb: (cb + cb_off, 0))],
        out_specs=pl.BlockSpec((1, b), lambda cb: (0, 0)),
        out_shape=jax.ShapeDtypeStruct((1, b), jnp.float32),
        scratch_shapes=[pltpu.VMEM((8, b), jnp.float32)],
        compiler_params=pltpu.CompilerParams(
            dimension_semantics=("arbitrary",),
        ),
    )(xt)

    losses = pl.pallas_call(
        _epilogue_kernel,
        in_specs=[
            pl.BlockSpec((1, b), lambda: (0, 0)),
            pl.BlockSpec((_NW, b), lambda: (0, 0)),
            pl.BlockSpec((1, b), lambda: (0, 0)),
            pl.BlockSpec((1, b), lambda: (0, 0)),
            pl.BlockSpec((1, b), lambda: (0, 0)),
        ],
        out_specs=pl.BlockSpec((1, b), lambda: (0, 0)),
        out_shape=jax.ShapeDtypeStruct((1, b), jnp.float32),
    )(tc_sums, sc_parts.reshape(_NW, b), sc_xlab[None, :], cos_m, sin_m)

    return jnp.mean(losses)
